# Initial kernel scaffold; baseline (speedup 1.0000x reference)
import jax
import jax.numpy as jnp
from jax.experimental import pallas as pl


def kernel(x, W1a, g1a, b1a, W1b, g1b, b1b, W2a, g2a, b2a, W2b, g2b, b2b):
    raise NotImplementedError("write your pallas kernel here")


# speedup vs baseline: 6.7725x; 6.7725x over previous
import functools

import jax
import jax.numpy as jnp
from jax import lax
from jax.experimental import pallas as pl
from jax.experimental.pallas import tpu as pltpu
from jax.experimental.pallas import tpu_sc as plsc

B, N, CIN, C, K = 4, 4096, 3, 128, 32
C2 = 2 * C
M1 = float(B * N)
M2 = float(B * N * K)
ROWS = B * N * K
RT = 256
NT = N // RT
GT = 1024
NG = ROWS // GT
NEG = float("-inf")


def _l1_body(x8_ref, w1a_ref, g1a_ref, b1a_ref, w1b_ref, g1b_ref, b1b_ref,
             h_ref, hsq_ref):
    def bn_relu(y, g, b):
        mean = jnp.sum(y, axis=0, keepdims=True) / M1
        var = jnp.sum(y * y, axis=0, keepdims=True) / M1 - mean * mean
        yh = (y - mean) * lax.rsqrt(var + 1e-5) * g + b
        return jnp.where(yh >= 0, yh, 0.01 * yh)

    y1 = jnp.dot(x8_ref[...], w1a_ref[...], preferred_element_type=jnp.float32)
    h1 = bn_relu(y1, g1a_ref[...], b1a_ref[...])
    y2 = jnp.dot(h1, w1b_ref[...], preferred_element_type=jnp.float32)
    h = bn_relu(y2, g1b_ref[...], b1b_ref[...])
    h_ref[...] = h
    hsq_ref[...] = jnp.sum(h * h, axis=1, keepdims=True)


def _layer1(x, W1a, g1a, b1a, W1b, g1b, b1b):
    x8 = jnp.pad(x.reshape(B * N, CIN), ((0, 0), (0, 8 - CIN)))
    w1a = jnp.pad(W1a.T, ((0, 8 - CIN), (0, 0)))
    return pl.pallas_call(
        _l1_body,
        out_shape=(jax.ShapeDtypeStruct((B * N, C), jnp.float32),
                   jax.ShapeDtypeStruct((B * N, 1), jnp.float32)),
    )(x8, w1a, g1a.reshape(1, C), b1a.reshape(1, C),
      W1b.T, g1b.reshape(1, C), b1b.reshape(1, C))


def _knn_body(h_ref, sq_ref, idx_ref):
    b = pl.program_id(0)
    t = pl.program_id(1)
    hb = h_ref[0]
    tile = h_ref[0, pl.ds(t * RT, RT), :]
    inner = lax.dot_general(tile, hb, (((1,), (1,)), ((), ())),
                            preferred_element_type=jnp.float32)
    score = 2.0 * inner - sq_ref[0]
    iota = lax.broadcasted_iota(jnp.int32, (RT, N), 1)
    cols = []
    d = score
    for _ in range(K):
        m = jnp.max(d, axis=1, keepdims=True)
        arg = jnp.min(jnp.where(d == m, iota, jnp.int32(N)),
                      axis=1, keepdims=True)
        cols.append(arg)
        d = jnp.where(iota == arg, NEG, d)
    idx_ref[0] = jnp.concatenate(cols, axis=1) + b * N


def _knn(h3, sqT):
    return pl.pallas_call(
        _knn_body,
        grid=(B, NT),
        in_specs=[
            pl.BlockSpec((1, N, C), lambda b, t: (b, 0, 0)),
            pl.BlockSpec((1, 1, N), lambda b, t: (b, 0, 0)),
        ],
        out_specs=pl.BlockSpec((1, RT, K), lambda b, t: (b, t, 0)),
        out_shape=jax.ShapeDtypeStruct((B, N, K), jnp.int32),
    )(h3, sqT)


def _fold_body(h_ref, wd_ref, wcd_ref, a_ref, bc_ref):
    h = h_ref[...]
    a_ref[...] = jnp.dot(h, wd_ref[...], preferred_element_type=jnp.float32)
    bc_ref[...] = jnp.dot(h, wcd_ref[...], preferred_element_type=jnp.float32)


def _fold(h, W2a):
    wd = W2a[:, :C].T
    wcd = (W2a[:, C:] - W2a[:, :C]).T
    return pl.pallas_call(
        _fold_body,
        grid=(16,),
        in_specs=[
            pl.BlockSpec((1024, C), lambda i: (i, 0)),
            pl.BlockSpec((C, C2), lambda i: (0, 0)),
            pl.BlockSpec((C, C2), lambda i: (0, 0)),
        ],
        out_specs=(pl.BlockSpec((1024, C2), lambda i: (i, 0)),
                   pl.BlockSpec((1024, C2), lambda i: (i, 0))),
        out_shape=(jax.ShapeDtypeStruct((B * N, C2), jnp.float32),
                   jax.ShapeDtypeStruct((B * N, C2), jnp.float32)),
    )(h, wd, wcd)


_NC, _NS = 2, 16
NW = _NC * _NS
RPW = ROWS // NW
CH = 128


def _sc_gather(A, fidx):
    mesh = plsc.VectorSubcoreMesh(core_axis_name="c", subcore_axis_name="s")

    @functools.partial(
        pl.kernel, mesh=mesh,
        out_type=jax.ShapeDtypeStruct((ROWS, C2), jnp.float32),
        scratch_types=[
            pltpu.VMEM((RPW,), jnp.int32),
            pltpu.VMEM((CH, C2), jnp.float32),
            pltpu.SemaphoreType.DMA,
        ],
    )
    def gather_k(a_hbm, idx_hbm, out_hbm, idx_v, rows_v, sem):
        wid = lax.axis_index("s") * _NC + lax.axis_index("c")
        base = wid * RPW
        pltpu.sync_copy(idx_hbm.at[pl.ds(base, RPW)], idx_v)

        def body(i, _):
            pltpu.async_copy(a_hbm.at[idx_v.at[pl.ds(i * CH, CH)]],
                             rows_v, sem).wait()
            pltpu.sync_copy(rows_v, out_hbm.at[pl.ds(base + i * CH, CH)])
            return 0

        lax.fori_loop(0, RPW // CH, body, 0)

    return gather_k(A, fidx)


def _stats_body(y0_ref, bc_ref, s_ref, ss_ref):
    g = pl.program_id(0)
    bc = bc_ref[...]
    y = y0_ref[...] + jnp.broadcast_to(
        bc[:, None, :], (GT // K, K, C2)).reshape(GT, C2)

    @pl.when(g == 0)
    def _():
        s_ref[...] = jnp.zeros_like(s_ref)
        ss_ref[...] = jnp.zeros_like(ss_ref)

    s_ref[...] += jnp.sum(y, axis=0, keepdims=True)
    ss_ref[...] += jnp.sum(y * y, axis=0, keepdims=True)


def _stats(y0, Bc):
    return pl.pallas_call(
        _stats_body,
        grid=(NG,),
        in_specs=[
            pl.BlockSpec((GT, C2), lambda g: (g, 0)),
            pl.BlockSpec((GT // K, C2), lambda g: (g, 0)),
        ],
        out_specs=(pl.BlockSpec((1, C2), lambda g: (0, 0)),
                   pl.BlockSpec((1, C2), lambda g: (0, 0))),
        out_shape=(jax.ShapeDtypeStruct((1, C2), jnp.float32),
                   jax.ShapeDtypeStruct((1, C2), jnp.float32)),
    )(y0, Bc)


def _main_body(y0_ref, bc_ref, sc_ref, sh_ref, w_ref,
               zmx_ref, zmn_ref, s_ref, ss_ref):
    g = pl.program_id(0)
    bc = bc_ref[...]
    y = y0_ref[...] + jnp.broadcast_to(
        bc[:, None, :], (GT // K, K, C2)).reshape(GT, C2)
    yh = y * sc_ref[...] + sh_ref[...]
    f = jnp.where(yh >= 0, yh, 0.01 * yh)
    z = jnp.dot(f, w_ref[...], preferred_element_type=jnp.float32)

    @pl.when(g == 0)
    def _():
        s_ref[...] = jnp.zeros_like(s_ref)
        ss_ref[...] = jnp.zeros_like(ss_ref)

    s_ref[...] += jnp.sum(z, axis=0, keepdims=True)
    ss_ref[...] += jnp.sum(z * z, axis=0, keepdims=True)
    z3 = z.reshape(GT // K, K, C2)
    zmx_ref[...] = jnp.max(z3, axis=1)
    zmn_ref[...] = jnp.min(z3, axis=1)


def _main(y0, Bc, scale_a, shift_a, W2b):
    return pl.pallas_call(
        _main_body,
        grid=(NG,),
        in_specs=[
            pl.BlockSpec((GT, C2), lambda g: (g, 0)),
            pl.BlockSpec((GT // K, C2), lambda g: (g, 0)),
            pl.BlockSpec((1, C2), lambda g: (0, 0)),
            pl.BlockSpec((1, C2), lambda g: (0, 0)),
            pl.BlockSpec((C2, C2), lambda g: (0, 0)),
        ],
        out_specs=(pl.BlockSpec((GT // K, C2), lambda g: (g, 0)),
                   pl.BlockSpec((GT // K, C2), lambda g: (g, 0)),
                   pl.BlockSpec((1, C2), lambda g: (0, 0)),
                   pl.BlockSpec((1, C2), lambda g: (0, 0))),
        out_shape=(jax.ShapeDtypeStruct((B * N, C2), jnp.float32),
                   jax.ShapeDtypeStruct((B * N, C2), jnp.float32),
                   jax.ShapeDtypeStruct((1, C2), jnp.float32),
                   jax.ShapeDtypeStruct((1, C2), jnp.float32)),
    )(y0, Bc, scale_a, shift_a, W2b.T)


def _fin_body(zmx_ref, zmn_ref, sc_ref, sh_ref, o_ref):
    sc = sc_ref[...]
    zz = jnp.where(sc >= 0, zmx_ref[...], zmn_ref[...])
    yh = zz * sc + sh_ref[...]
    o_ref[...] = jnp.where(yh >= 0, yh, 0.01 * yh)


def _final(zmx, zmn, scale_b, shift_b):
    return pl.pallas_call(
        _fin_body,
        grid=(16,),
        in_specs=[
            pl.BlockSpec((1024, C2), lambda i: (i, 0)),
            pl.BlockSpec((1024, C2), lambda i: (i, 0)),
            pl.BlockSpec((1, C2), lambda i: (0, 0)),
            pl.BlockSpec((1, C2), lambda i: (0, 0)),
        ],
        out_specs=pl.BlockSpec((1024, C2), lambda i: (i, 0)),
        out_shape=jax.ShapeDtypeStruct((B * N, C2), jnp.float32),
    )(zmx, zmn, scale_b, shift_b)


def kernel(x, W1a, g1a, b1a, W1b, g1b, b1b, W2a, g2a, b2a, W2b, g2b, b2b):
    h, hsq = _layer1(x, W1a, g1a, b1a, W1b, g1b, b1b)
    idx = _knn(h.reshape(B, N, C), hsq.reshape(B, 1, N))
    A, Bc = _fold(h, W2a)
    y0 = _sc_gather(A, idx.reshape(ROWS))
    s_a, ss_a = _stats(y0, Bc)

    mean_a = s_a / M2
    var_a = ss_a / M2 - mean_a * mean_a
    scale_a = g2a.reshape(1, C2) * lax.rsqrt(var_a + 1e-5)
    shift_a = b2a.reshape(1, C2) - mean_a * scale_a

    zmx, zmn, s_b, ss_b = _main(y0, Bc, scale_a, shift_a, W2b)

    mean_b = s_b / M2
    var_b = ss_b / M2 - mean_b * mean_b
    scale_b = g2b.reshape(1, C2) * lax.rsqrt(var_b + 1e-5)
    shift_b = b2b.reshape(1, C2) - mean_b * scale_b

    out = _final(zmx, zmn, scale_b, shift_b)
    return out.reshape(B, N, C2)
